# baseline (speedup 1.0000x reference)
import functools

import jax
import jax.numpy as jnp
from jax import lax
from jax.experimental import pallas as pl
from jax.experimental.pallas import tpu as pltpu
from jax.experimental.pallas import tpu_sc as plsc

B = 1024
N_SLOTS = 16
VAR = 512
D_QK = 32
N_RULES = 8
EMB = 64
HID = 1024
D_IN = 1024
N_STAGES = 4

BLK_B = 128
NB = B // BLK_B
BLK_P = 128
NPAD = B + N_RULES * BLK_P
NBLK_MLP = NPAD // BLK_P

SC_CORES = 2
SC_SUBCORES = 16
SC_WORKERS = SC_CORES * SC_SUBCORES
ROWS_PER_W = B // SC_WORKERS

_SCALE = 0.1767766952966369


def _bf(x):
    return x.astype(jnp.bfloat16)


def _dot(a, b):
    return jnp.dot(_bf(a), _bf(b), preferred_element_type=jnp.float32)


def _enc_body(x_ref, w_ref, b_ref, vars_ref, v_ref):
    enc = jnp.dot(_bf(x_ref[...]), w_ref[...],
                  preferred_element_type=jnp.float32) + b_ref[...]
    v_ref[...] = vars_ref[...] + enc[:, None, :]


def _encoder(x, vars0, W_enc_b, b_enc):
    return pl.pallas_call(
        _enc_body,
        grid=(NB,),
        in_specs=[
            pl.BlockSpec((BLK_B, D_IN), lambda i: (i, 0)),
            pl.BlockSpec((D_IN, VAR), lambda i: (0, 0)),
            pl.BlockSpec((1, VAR), lambda i: (0, 0)),
            pl.BlockSpec((BLK_B, N_SLOTS, VAR), lambda i: (i, 0, 0)),
        ],
        out_specs=pl.BlockSpec((BLK_B, N_SLOTS, VAR), lambda i: (i, 0, 0)),
        out_shape=jax.ShapeDtypeStruct((B, N_SLOTS, VAR), jnp.float32),
    )(x, W_enc_b, b_enc, vars0)


def _argmax_lanes(mat, n):
    m = jnp.max(mat, axis=1, keepdims=True)
    ii = lax.broadcasted_iota(jnp.int32, mat.shape, 1)
    return jnp.min(jnp.where(mat == m, ii, n), axis=1, keepdims=True)


def _sel_body(apply_prev, *refs):
    if apply_prev:
        (v_ref, vnew_ref, sprev_ref, wqvr_ref, wkvr_ref, remb_ref, wqpc_ref,
         wkpc_ref, vout_ref, inp_ref, sstar_ref, rstar_ref) = refs
        sprev = sprev_ref[...]
        vnew = vnew_ref[...]
        vs = []
        for s in range(N_SLOTS):
            v_s = jnp.where(sprev == s, vnew, v_ref[:, s, :])
            vout_ref[:, s, :] = v_s
            vs.append(v_s)
    else:
        (v_ref, wqvr_ref, wkvr_ref, remb_ref, wqpc_ref, wkpc_ref,
         inp_ref, sstar_ref, rstar_ref) = refs
        vs = [v_ref[:, s, :] for s in range(N_SLOTS)]

    k = jnp.dot(_bf(remb_ref[...]), wkvr_ref[...],
                preferred_element_type=jnp.float32)
    kb = _bf(k)
    logit_parts = []
    kc_parts = []
    for s in range(N_SLOTS):
        vs_b = _bf(vs[s])
        q_s = jnp.dot(vs_b, wqvr_ref[...],
                      preferred_element_type=jnp.float32)
        logit_parts.append(
            lax.dot_general(_bf(q_s), kb, (((1,), (1,)), ((), ())),
                            preferred_element_type=jnp.float32))
        kc_parts.append(jnp.dot(vs_b, wkpc_ref[...],
                                preferred_element_type=jnp.float32))
    flat = jnp.concatenate(logit_parts, axis=1) * _SCALE
    idx = _argmax_lanes(flat, N_SLOTS * N_RULES)
    sstar = idx >> 3
    rstar = idx & (N_RULES - 1)

    var_primary = jnp.zeros((BLK_B, VAR), jnp.float32)
    for s in range(N_SLOTS):
        var_primary = var_primary + jnp.where(sstar == s, vs[s], 0.0)

    qp = jnp.dot(_bf(var_primary), wqpc_ref[...],
                 preferred_element_type=jnp.float32)
    qpf = _bf(qp).astype(jnp.float32)
    ctx_cols = [
        jnp.sum(qpf * _bf(kc_parts[s]).astype(jnp.float32),
                axis=1, keepdims=True)
        for s in range(N_SLOTS)
    ]
    ctx_logits = jnp.concatenate(ctx_cols, axis=1) * _SCALE
    cidx = _argmax_lanes(ctx_logits, N_SLOTS)
    var_context = jnp.zeros((BLK_B, VAR), jnp.float32)
    for s in range(N_SLOTS):
        var_context = var_context + jnp.where(cidx == s, vs[s], 0.0)

    inp_ref[...] = jnp.concatenate([var_primary, var_context], axis=1)
    sstar_ref[...] = sstar
    rstar_ref[...] = rstar


def _selection(v, weights, prev=None):
    Wq_vr_b, Wk_vr_b, rule_emb, Wq_pc_b, Wk_pc_b = weights
    apply_prev = prev is not None
    wspecs = [
        pl.BlockSpec((VAR, D_QK), lambda i: (0, 0)),
        pl.BlockSpec((EMB, D_QK), lambda i: (0, 0)),
        pl.BlockSpec((N_RULES, EMB), lambda i: (0, 0)),
        pl.BlockSpec((VAR, D_QK), lambda i: (0, 0)),
        pl.BlockSpec((VAR, D_QK), lambda i: (0, 0)),
    ]
    in_specs = [pl.BlockSpec((BLK_B, N_SLOTS, VAR), lambda i: (i, 0, 0))]
    args = [v]
    if apply_prev:
        vnew, sprev = prev
        in_specs += [
            pl.BlockSpec((BLK_B, VAR), lambda i: (i, 0)),
            pl.BlockSpec((BLK_B, 1), lambda i: (i, 0)),
        ]
        args += [vnew, sprev]
    in_specs += wspecs
    args += [Wq_vr_b, Wk_vr_b, rule_emb, Wq_pc_b, Wk_pc_b]

    out_specs = [
        pl.BlockSpec((BLK_B, 2 * VAR), lambda i: (i, 0)),
        pl.BlockSpec((BLK_B, 1), lambda i: (i, 0)),
        pl.BlockSpec((BLK_B, 1), lambda i: (i, 0)),
    ]
    out_shape = [
        jax.ShapeDtypeStruct((B, 2 * VAR), jnp.float32),
        jax.ShapeDtypeStruct((B, 1), jnp.int32),
        jax.ShapeDtypeStruct((B, 1), jnp.int32),
    ]
    if apply_prev:
        out_specs = [pl.BlockSpec((BLK_B, N_SLOTS, VAR),
                                  lambda i: (i, 0, 0))] + out_specs
        out_shape = [jax.ShapeDtypeStruct((B, N_SLOTS, VAR),
                                          jnp.float32)] + out_shape

    res = pl.pallas_call(
        functools.partial(_sel_body, apply_prev),
        grid=(NB,),
        in_specs=in_specs,
        out_specs=out_specs,
        out_shape=out_shape,
    )(*args)
    if apply_prev:
        vnext, inp, sstar, rstar = res
    else:
        inp, sstar, rstar = res
        vnext = v
    return vnext, inp, sstar, rstar


def _route_body(rstar_ref, pos_ref, brule_ref):
    r = rstar_ref[...]
    rr = lax.broadcasted_iota(jnp.int32, (B, N_RULES), 1)
    oh = (r == rr).astype(jnp.float32)
    bi = lax.broadcasted_iota(jnp.int32, (B, B), 0)
    bj = lax.broadcasted_iota(jnp.int32, (B, B), 1)
    lower = _bf((bi >= bj).astype(jnp.float32))
    pref = jnp.dot(lower, _bf(oh), preferred_element_type=jnp.float32)
    rank = jnp.sum(oh * pref, axis=1, keepdims=True) - 1.0
    counts = pref[B - 1:B, :]
    nblk = (counts.astype(jnp.int32) + (BLK_P - 1)) // BLK_P
    ri = lax.broadcasted_iota(jnp.int32, (N_RULES, N_RULES), 0)
    rj = lax.broadcasted_iota(jnp.int32, (N_RULES, N_RULES), 1)
    supper = _bf((ri < rj).astype(jnp.float32))
    blkoff = jnp.dot(_bf(nblk.astype(jnp.float32)), supper,
                     preferred_element_type=jnp.float32)
    start = jnp.sum(oh * blkoff, axis=1, keepdims=True)
    pos_ref[...] = (start * BLK_P + rank).astype(jnp.int32)
    jcol = lax.broadcasted_iota(jnp.int32, (NBLK_MLP, N_RULES), 0)
    bo = jnp.broadcast_to(blkoff, (NBLK_MLP, N_RULES))
    cnt = jnp.sum((bo <= jcol.astype(jnp.float32)).astype(jnp.float32),
                  axis=1, keepdims=True)
    brule_ref[...] = cnt.astype(jnp.int32) - 1


def _routing(rstar):
    return pl.pallas_call(
        _route_body,
        grid=(1,),
        in_specs=[pl.BlockSpec((B, 1), lambda i: (0, 0))],
        out_specs=[
            pl.BlockSpec((B, 1), lambda i: (0, 0)),
            pl.BlockSpec((NBLK_MLP, 1), lambda i: (0, 0)),
        ],
        out_shape=[
            jax.ShapeDtypeStruct((B, 1), jnp.int32),
            jax.ShapeDtypeStruct((NBLK_MLP, 1), jnp.int32),
        ],
    )(rstar)


@functools.cache
def _sc_scatter_kernel():
    mesh = plsc.VectorSubcoreMesh(core_axis_name="c", subcore_axis_name="s")

    @functools.partial(
        pl.kernel,
        mesh=mesh,
        out_type=jax.ShapeDtypeStruct((NPAD, 2 * VAR), jnp.float32),
        scratch_types=[
            pltpu.VMEM((ROWS_PER_W,), jnp.int32),
            pltpu.VMEM((ROWS_PER_W, 2 * VAR), jnp.float32),
            pltpu.SemaphoreType.DMA,
        ],
    )
    def body(inp_hbm, pos_hbm, out_hbm, pos_v, rows_v, sem):
        wid = lax.axis_index("s") * SC_CORES + lax.axis_index("c")
        base = wid * ROWS_PER_W
        pltpu.sync_copy(pos_hbm.at[pl.ds(base, ROWS_PER_W)], pos_v)
        pltpu.sync_copy(inp_hbm.at[pl.ds(base, ROWS_PER_W)], rows_v)
        pltpu.async_copy(rows_v, out_hbm.at[pos_v], sem).wait()

    return body


def _sc_scatter(inp, pos):
    return _sc_scatter_kernel()(inp, pos)


@functools.cache
def _sc_gather_kernel():
    mesh = plsc.VectorSubcoreMesh(core_axis_name="c", subcore_axis_name="s")

    @functools.partial(
        pl.kernel,
        mesh=mesh,
        out_type=jax.ShapeDtypeStruct((B, VAR), jnp.float32),
        scratch_types=[
            pltpu.VMEM((ROWS_PER_W,), jnp.int32),
            pltpu.VMEM((ROWS_PER_W, VAR), jnp.float32),
            pltpu.SemaphoreType.DMA,
        ],
    )
    def body(outsrt_hbm, pos_hbm, vnew_hbm, pos_v, rows_v, sem):
        wid = lax.axis_index("s") * SC_CORES + lax.axis_index("c")
        base = wid * ROWS_PER_W
        pltpu.sync_copy(pos_hbm.at[pl.ds(base, ROWS_PER_W)], pos_v)
        pltpu.async_copy(outsrt_hbm.at[pos_v], rows_v, sem).wait()
        pltpu.sync_copy(rows_v, vnew_hbm.at[pl.ds(base, ROWS_PER_W)])

    return body


def _sc_gather(outsrt, pos):
    return _sc_gather_kernel()(outsrt, pos)


def _mlp_body(brule_ref, inp_ref, w1_ref, b1_ref, w2_ref, b2_ref, out_ref):
    del brule_ref
    h = jnp.dot(_bf(inp_ref[...]), w1_ref[0],
                preferred_element_type=jnp.float32) + b1_ref[0]
    h = jnp.maximum(h, 0.0)
    out_ref[...] = jnp.dot(_bf(h), w2_ref[0],
                           preferred_element_type=jnp.float32) + b2_ref[0]


def _grouped_mlp(inp_sorted, brule, W1_b, b1, W2_b, b2):
    grid_spec = pltpu.PrefetchScalarGridSpec(
        num_scalar_prefetch=1,
        grid=(NBLK_MLP,),
        in_specs=[
            pl.BlockSpec((BLK_P, 2 * VAR), lambda j, br: (j, 0)),
            pl.BlockSpec((1, 2 * VAR, HID), lambda j, br: (br[j], 0, 0)),
            pl.BlockSpec((1, 1, HID), lambda j, br: (br[j], 0, 0)),
            pl.BlockSpec((1, HID, VAR), lambda j, br: (br[j], 0, 0)),
            pl.BlockSpec((1, 1, VAR), lambda j, br: (br[j], 0, 0)),
        ],
        out_specs=pl.BlockSpec((BLK_P, VAR), lambda j, br: (j, 0)),
    )
    return pl.pallas_call(
        _mlp_body,
        grid_spec=grid_spec,
        out_shape=jax.ShapeDtypeStruct((NPAD, VAR), jnp.float32),
    )(brule, inp_sorted, W1_b, b1, W2_b, b2)


def _apply_body(v_ref, vnew_ref, sprev_ref, vout_ref):
    sprev = sprev_ref[...]
    vnew = vnew_ref[...]
    for s in range(N_SLOTS):
        vout_ref[:, s, :] = jnp.where(sprev == s, vnew, v_ref[:, s, :])


def _apply(v, vnew, sstar):
    return pl.pallas_call(
        _apply_body,
        grid=(NB,),
        in_specs=[
            pl.BlockSpec((BLK_B, N_SLOTS, VAR), lambda i: (i, 0, 0)),
            pl.BlockSpec((BLK_B, VAR), lambda i: (i, 0)),
            pl.BlockSpec((BLK_B, 1), lambda i: (i, 0)),
        ],
        out_specs=pl.BlockSpec((BLK_B, N_SLOTS, VAR), lambda i: (i, 0, 0)),
        out_shape=jax.ShapeDtypeStruct((B, N_SLOTS, VAR), jnp.float32),
    )(v, vnew, sstar)


def kernel(x, vars0, W_enc, b_enc, Wq_vr, Wk_vr, Wq_pc, Wk_pc, rule_emb,
           rule_W1, rule_b1, rule_W2, rule_b2):
    W_enc_b = _bf(W_enc)
    Wq_vr_b = _bf(Wq_vr)
    Wk_vr_b = _bf(Wk_vr)
    Wq_pc_b = _bf(Wq_pc)
    Wk_pc_b = _bf(Wk_pc)
    W1_b = _bf(rule_W1)
    W2_b = _bf(rule_W2)
    sel_w = (Wq_vr_b, Wk_vr_b, rule_emb, Wq_pc_b, Wk_pc_b)

    v = _encoder(x, vars0, W_enc_b, b_enc.reshape(1, VAR))
    prev = None
    for _ in range(N_STAGES):
        v, inp, sstar, rstar = _selection(v, sel_w, prev)
        pos, brule = _routing(rstar)
        pos1 = pos.reshape(B)
        inp_sorted = _sc_scatter(inp, pos1)
        out_sorted = _grouped_mlp(inp_sorted, brule.reshape(NBLK_MLP),
                                  W1_b, rule_b1.reshape(N_RULES, 1, HID),
                                  W2_b, rule_b2.reshape(N_RULES, 1, VAR))
        vnew = _sc_gather(out_sorted, pos1)
        prev = (vnew, sstar)
    return _apply(v, prev[0], prev[1])


# speedup vs baseline: 1.2178x; 1.2178x over previous
import functools

import jax
import jax.numpy as jnp
from jax import lax
from jax.experimental import pallas as pl
from jax.experimental.pallas import tpu as pltpu
from jax.experimental.pallas import tpu_sc as plsc

B = 1024
N_SLOTS = 16
VAR = 512
D_QK = 32
N_RULES = 8
EMB = 64
HID = 1024
D_IN = 1024
N_STAGES = 4

BLK_B = 128
NB = B // BLK_B
BLK_P = 128
NPAD = B + N_RULES * BLK_P
NBLK_MLP = NPAD // BLK_P

SC_CORES = 2
SC_SUBCORES = 16
SC_WORKERS = SC_CORES * SC_SUBCORES
ROWS_PER_W = B // SC_WORKERS

_SCALE = 0.1767766952966369


def _bf(x):
    return x.astype(jnp.bfloat16)


def _dot(a, b):
    return jnp.dot(_bf(a), _bf(b), preferred_element_type=jnp.float32)


def _enc_body(x_ref, w_ref, b_ref, vars_ref, v_ref):
    enc = jnp.dot(_bf(x_ref[...]), w_ref[...],
                  preferred_element_type=jnp.float32) + b_ref[...]
    v_ref[...] = vars_ref[...] + enc[:, None, :]


def _encoder(x, vars0, W_enc_b, b_enc):
    return pl.pallas_call(
        _enc_body,
        grid=(NB,),
        in_specs=[
            pl.BlockSpec((BLK_B, D_IN), lambda i: (i, 0)),
            pl.BlockSpec((D_IN, VAR), lambda i: (0, 0)),
            pl.BlockSpec((1, VAR), lambda i: (0, 0)),
            pl.BlockSpec((BLK_B, N_SLOTS, VAR), lambda i: (i, 0, 0)),
        ],
        out_specs=pl.BlockSpec((BLK_B, N_SLOTS, VAR), lambda i: (i, 0, 0)),
        out_shape=jax.ShapeDtypeStruct((B, N_SLOTS, VAR), jnp.float32),
    )(x, W_enc_b, b_enc, vars0)


def _argmax_lanes(mat, n):
    m = jnp.max(mat, axis=1, keepdims=True)
    ii = lax.broadcasted_iota(jnp.int32, mat.shape, 1)
    return jnp.min(jnp.where(mat == m, ii, n), axis=1, keepdims=True)


def _sel_body(apply_prev, *refs):
    if apply_prev:
        (v_ref, vnew_ref, sprev_ref, wqkc_ref, wkvr_ref, remb_ref, wqpc_ref,
         vout_ref, inp_ref, sstar_ref, rstar_ref) = refs
        sprev = sprev_ref[...]
        vnew = vnew_ref[...]
        vs = []
        for s in range(N_SLOTS):
            v_s = jnp.where(sprev == s, vnew, v_ref[:, s, :])
            vout_ref[:, s, :] = v_s
            vs.append(v_s)
    else:
        (v_ref, wqkc_ref, wkvr_ref, remb_ref, wqpc_ref,
         inp_ref, sstar_ref, rstar_ref) = refs
        vs = [v_ref[:, s, :] for s in range(N_SLOTS)]

    vstack = _bf(jnp.concatenate(vs, axis=0))
    qkc = jnp.dot(vstack, wqkc_ref[...],
                  preferred_element_type=jnp.float32)
    k = jnp.dot(_bf(remb_ref[...]), wkvr_ref[...],
                preferred_element_type=jnp.float32)
    logits_all = lax.dot_general(_bf(qkc[:, :D_QK]), _bf(k),
                                 (((1,), (1,)), ((), ())),
                                 preferred_element_type=jnp.float32)
    flat = jnp.concatenate(
        [logits_all[s * BLK_B:(s + 1) * BLK_B, :] for s in range(N_SLOTS)],
        axis=1) * _SCALE
    idx = _argmax_lanes(flat, N_SLOTS * N_RULES)
    sstar = idx >> 3
    rstar = idx & (N_RULES - 1)

    var_primary = jnp.zeros((BLK_B, VAR), jnp.float32)
    for s in range(N_SLOTS):
        var_primary = var_primary + jnp.where(sstar == s, vs[s], 0.0)

    qp = jnp.dot(_bf(var_primary), wqpc_ref[...],
                 preferred_element_type=jnp.float32)
    qpf = _bf(qp).astype(jnp.float32)
    ctx_cols = [
        jnp.sum(qpf * _bf(qkc[s * BLK_B:(s + 1) * BLK_B, D_QK:])
                .astype(jnp.float32), axis=1, keepdims=True)
        for s in range(N_SLOTS)
    ]
    ctx_logits = jnp.concatenate(ctx_cols, axis=1) * _SCALE
    cidx = _argmax_lanes(ctx_logits, N_SLOTS)
    var_context = jnp.zeros((BLK_B, VAR), jnp.float32)
    for s in range(N_SLOTS):
        var_context = var_context + jnp.where(cidx == s, vs[s], 0.0)

    inp_ref[...] = jnp.concatenate([var_primary, var_context], axis=1)
    sstar_ref[...] = sstar
    rstar_ref[...] = rstar


def _selection(v, weights, prev=None):
    Wqkc_b, Wk_vr_b, rule_emb, Wq_pc_b = weights
    apply_prev = prev is not None
    wspecs = [
        pl.BlockSpec((VAR, 2 * D_QK), lambda i: (0, 0)),
        pl.BlockSpec((EMB, D_QK), lambda i: (0, 0)),
        pl.BlockSpec((N_RULES, EMB), lambda i: (0, 0)),
        pl.BlockSpec((VAR, D_QK), lambda i: (0, 0)),
    ]
    in_specs = [pl.BlockSpec((BLK_B, N_SLOTS, VAR), lambda i: (i, 0, 0))]
    args = [v]
    if apply_prev:
        vnew, sprev = prev
        in_specs += [
            pl.BlockSpec((BLK_B, VAR), lambda i: (i, 0)),
            pl.BlockSpec((BLK_B, 1), lambda i: (i, 0)),
        ]
        args += [vnew, sprev]
    in_specs += wspecs
    args += [Wqkc_b, Wk_vr_b, rule_emb, Wq_pc_b]

    out_specs = [
        pl.BlockSpec((BLK_B, 2 * VAR), lambda i: (i, 0)),
        pl.BlockSpec((BLK_B, 1), lambda i: (i, 0)),
        pl.BlockSpec((BLK_B, 1), lambda i: (i, 0)),
    ]
    out_shape = [
        jax.ShapeDtypeStruct((B, 2 * VAR), jnp.float32),
        jax.ShapeDtypeStruct((B, 1), jnp.int32),
        jax.ShapeDtypeStruct((B, 1), jnp.int32),
    ]
    if apply_prev:
        out_specs = [pl.BlockSpec((BLK_B, N_SLOTS, VAR),
                                  lambda i: (i, 0, 0))] + out_specs
        out_shape = [jax.ShapeDtypeStruct((B, N_SLOTS, VAR),
                                          jnp.float32)] + out_shape

    res = pl.pallas_call(
        functools.partial(_sel_body, apply_prev),
        grid=(NB,),
        in_specs=in_specs,
        out_specs=out_specs,
        out_shape=out_shape,
    )(*args)
    if apply_prev:
        vnext, inp, sstar, rstar = res
    else:
        inp, sstar, rstar = res
        vnext = v
    return vnext, inp, sstar, rstar


def _route_body(rstar_ref, pos_ref, brule_ref, nact_ref):
    r = rstar_ref[...]
    rr = lax.broadcasted_iota(jnp.int32, (B, N_RULES), 1)
    oh = (r == rr).astype(jnp.float32)
    bi = lax.broadcasted_iota(jnp.int32, (B, B), 0)
    bj = lax.broadcasted_iota(jnp.int32, (B, B), 1)
    lower = _bf((bi >= bj).astype(jnp.float32))
    pref = jnp.dot(lower, _bf(oh), preferred_element_type=jnp.float32)
    rank = jnp.sum(oh * pref, axis=1, keepdims=True) - 1.0
    counts = pref[B - 1:B, :]
    nblk = (counts.astype(jnp.int32) + (BLK_P - 1)) // BLK_P
    ri = lax.broadcasted_iota(jnp.int32, (N_RULES, N_RULES), 0)
    rj = lax.broadcasted_iota(jnp.int32, (N_RULES, N_RULES), 1)
    supper = _bf((ri < rj).astype(jnp.float32))
    blkoff = jnp.dot(_bf(nblk.astype(jnp.float32)), supper,
                     preferred_element_type=jnp.float32)
    start = jnp.sum(oh * blkoff, axis=1, keepdims=True)
    pos_ref[...] = (start * BLK_P + rank).astype(jnp.int32)
    total = jnp.sum(nblk.astype(jnp.float32), axis=1, keepdims=True)
    nact_ref[...] = total.astype(jnp.int32)
    jcol = lax.broadcasted_iota(jnp.int32, (NBLK_MLP, N_RULES), 0)
    jclamp = jnp.minimum(jcol.astype(jnp.float32),
                         jnp.broadcast_to(total, (NBLK_MLP, N_RULES)) - 1.0)
    bo = jnp.broadcast_to(blkoff, (NBLK_MLP, N_RULES))
    cnt = jnp.sum((bo <= jclamp).astype(jnp.float32),
                  axis=1, keepdims=True)
    brule_ref[...] = cnt.astype(jnp.int32) - 1


def _routing(rstar):
    return pl.pallas_call(
        _route_body,
        grid=(1,),
        in_specs=[pl.BlockSpec((B, 1), lambda i: (0, 0))],
        out_specs=[
            pl.BlockSpec((B, 1), lambda i: (0, 0)),
            pl.BlockSpec((NBLK_MLP, 1), lambda i: (0, 0)),
            pl.BlockSpec((1, 1), lambda i: (0, 0)),
        ],
        out_shape=[
            jax.ShapeDtypeStruct((B, 1), jnp.int32),
            jax.ShapeDtypeStruct((NBLK_MLP, 1), jnp.int32),
            jax.ShapeDtypeStruct((1, 1), jnp.int32),
        ],
    )(rstar)


@functools.cache
def _sc_scatter_kernel():
    mesh = plsc.VectorSubcoreMesh(core_axis_name="c", subcore_axis_name="s")

    @functools.partial(
        pl.kernel,
        mesh=mesh,
        out_type=jax.ShapeDtypeStruct((NPAD, 2 * VAR), jnp.float32),
        scratch_types=[
            pltpu.VMEM((ROWS_PER_W,), jnp.int32),
            pltpu.VMEM((ROWS_PER_W, 2 * VAR), jnp.float32),
            pltpu.SemaphoreType.DMA,
        ],
    )
    def body(inp_hbm, pos_hbm, out_hbm, pos_v, rows_v, sem):
        wid = lax.axis_index("s") * SC_CORES + lax.axis_index("c")
        base = wid * ROWS_PER_W
        pltpu.sync_copy(pos_hbm.at[pl.ds(base, ROWS_PER_W)], pos_v)
        pltpu.sync_copy(inp_hbm.at[pl.ds(base, ROWS_PER_W)], rows_v)
        pltpu.async_copy(rows_v, out_hbm.at[pos_v], sem).wait()

    return body


def _sc_scatter(inp, pos):
    return _sc_scatter_kernel()(inp, pos)


@functools.cache
def _sc_gather_kernel():
    mesh = plsc.VectorSubcoreMesh(core_axis_name="c", subcore_axis_name="s")

    @functools.partial(
        pl.kernel,
        mesh=mesh,
        out_type=jax.ShapeDtypeStruct((B, VAR), jnp.float32),
        scratch_types=[
            pltpu.VMEM((ROWS_PER_W,), jnp.int32),
            pltpu.VMEM((ROWS_PER_W, VAR), jnp.float32),
            pltpu.SemaphoreType.DMA,
        ],
    )
    def body(outsrt_hbm, pos_hbm, vnew_hbm, pos_v, rows_v, sem):
        wid = lax.axis_index("s") * SC_CORES + lax.axis_index("c")
        base = wid * ROWS_PER_W
        pltpu.sync_copy(pos_hbm.at[pl.ds(base, ROWS_PER_W)], pos_v)
        pltpu.async_copy(outsrt_hbm.at[pos_v], rows_v, sem).wait()
        pltpu.sync_copy(rows_v, vnew_hbm.at[pl.ds(base, ROWS_PER_W)])

    return body


def _sc_gather(outsrt, pos):
    return _sc_gather_kernel()(outsrt, pos)


def _mlp_body(brule_ref, nact_ref, inp_ref, w1_ref, b1_ref, w2_ref, b2_ref,
              out_ref):
    del brule_ref

    @pl.when(pl.program_id(0) < nact_ref[0])
    def _():
        h = jnp.dot(_bf(inp_ref[...]), w1_ref[0],
                    preferred_element_type=jnp.float32) + b1_ref[0]
        h = jnp.maximum(h, 0.0)
        out_ref[...] = jnp.dot(_bf(h), w2_ref[0],
                               preferred_element_type=jnp.float32) + b2_ref[0]


def _grouped_mlp(inp_sorted, brule, nact, W1_b, b1, W2_b, b2):
    grid_spec = pltpu.PrefetchScalarGridSpec(
        num_scalar_prefetch=2,
        grid=(NBLK_MLP,),
        in_specs=[
            pl.BlockSpec((BLK_P, 2 * VAR), lambda j, br, na: (j, 0)),
            pl.BlockSpec((1, 2 * VAR, HID), lambda j, br, na: (br[j], 0, 0)),
            pl.BlockSpec((1, 1, HID), lambda j, br, na: (br[j], 0, 0)),
            pl.BlockSpec((1, HID, VAR), lambda j, br, na: (br[j], 0, 0)),
            pl.BlockSpec((1, 1, VAR), lambda j, br, na: (br[j], 0, 0)),
        ],
        out_specs=pl.BlockSpec((BLK_P, VAR), lambda j, br, na: (j, 0)),
    )
    return pl.pallas_call(
        _mlp_body,
        grid_spec=grid_spec,
        out_shape=jax.ShapeDtypeStruct((NPAD, VAR), jnp.float32),
    )(brule, nact, inp_sorted, W1_b, b1, W2_b, b2)


def _apply_body(v_ref, vnew_ref, sprev_ref, vout_ref):
    sprev = sprev_ref[...]
    vnew = vnew_ref[...]
    for s in range(N_SLOTS):
        vout_ref[:, s, :] = jnp.where(sprev == s, vnew, v_ref[:, s, :])


def _apply(v, vnew, sstar):
    return pl.pallas_call(
        _apply_body,
        grid=(NB,),
        in_specs=[
            pl.BlockSpec((BLK_B, N_SLOTS, VAR), lambda i: (i, 0, 0)),
            pl.BlockSpec((BLK_B, VAR), lambda i: (i, 0)),
            pl.BlockSpec((BLK_B, 1), lambda i: (i, 0)),
        ],
        out_specs=pl.BlockSpec((BLK_B, N_SLOTS, VAR), lambda i: (i, 0, 0)),
        out_shape=jax.ShapeDtypeStruct((B, N_SLOTS, VAR), jnp.float32),
    )(v, vnew, sstar)


def kernel(x, vars0, W_enc, b_enc, Wq_vr, Wk_vr, Wq_pc, Wk_pc, rule_emb,
           rule_W1, rule_b1, rule_W2, rule_b2):
    W_enc_b = _bf(W_enc)
    Wqkc_b = jnp.concatenate([_bf(Wq_vr), _bf(Wk_pc)], axis=1)
    Wk_vr_b = _bf(Wk_vr)
    Wq_pc_b = _bf(Wq_pc)
    W1_b = _bf(rule_W1)
    W2_b = _bf(rule_W2)
    sel_w = (Wqkc_b, Wk_vr_b, rule_emb, Wq_pc_b)

    v = _encoder(x, vars0, W_enc_b, b_enc.reshape(1, VAR))
    prev = None
    for _ in range(N_STAGES):
        v, inp, sstar, rstar = _selection(v, sel_w, prev)
        pos, brule, nact = _routing(rstar)
        pos1 = pos.reshape(B)
        inp_sorted = _sc_scatter(inp, pos1)
        out_sorted = _grouped_mlp(inp_sorted, brule.reshape(NBLK_MLP),
                                  nact.reshape(1),
                                  W1_b, rule_b1.reshape(N_RULES, 1, HID),
                                  W2_b, rule_b2.reshape(N_RULES, 1, VAR))
        vnew = _sc_gather(out_sorted, pos1)
        prev = (vnew, sstar)
    return _apply(v, prev[0], prev[1])
